# baseline (speedup 1.0000x reference)
import functools

import jax
import jax.numpy as jnp
from jax import lax
from jax.experimental import pallas as pl
from jax.experimental.pallas import tpu as pltpu
from jax.experimental.pallas import tpu_sc as plsc

B, N, M, C, K = 4, 1024, 8192, 512, 16
MC = 8
MT = M // MC
SEG = 128
G = M // SEG
GPC = MT // SEG
CAND = K * SEG


def _sim_body(t_ref, r_ref, tss_ref, rss_ref, sim_ref, seg_ref, gv_s):
    b = pl.program_id(0)
    mc = pl.program_id(1)

    t = t_ref[0]
    r = r_ref[0]
    tn = t / (jnp.sqrt(tss_ref[0]) + 1e-8)
    rn = r / (jnp.sqrt(rss_ref[0]) + 1e-8)
    for g in range(GPC):
        sim_g = lax.dot_general(tn, rn[g * SEG:(g + 1) * SEG],
                                (((1,), (1,)), ((), ())),
                                preferred_element_type=jnp.float32)
        sim_ref[0, g] = sim_g
        gv_s[mc * GPC + g, :] = jnp.max(sim_g, axis=1)

    @pl.when(mc == MC - 1)
    def _final():
        vcur = gv_s[...]
        giota = lax.broadcasted_iota(jnp.int32, (G, N), 0)
        picks = []
        for _ in range(K):
            m = jnp.max(vcur, axis=0)
            ismax = vcur == m[None, :]
            pos = jnp.min(jnp.where(ismax, giota, G), axis=0)
            picks.append(pos)
            vcur = jnp.where(giota == pos[None, :], -jnp.inf, vcur)
        outs = []
        big = jnp.int32(G)
        for _ in range(K):
            mn = picks[0]
            for p in picks[1:]:
                mn = jnp.minimum(mn, p)
            outs.append(mn)
            picks = [jnp.where(p == mn, big, p) for p in picks]
        niota = lax.broadcasted_iota(jnp.int32, (N, K), 0)
        seg_ref[0] = jnp.stack(outs, axis=1) * N + niota + b * (G * N)


_simtop = pl.pallas_call(
    _sim_body,
    grid=(B, MC),
    in_specs=[
        pl.BlockSpec((1, N, C), lambda b, mc: (b, 0, 0)),
        pl.BlockSpec((1, MT, C), lambda b, mc: (b, mc, 0)),
        pl.BlockSpec((1, N, 1), lambda b, mc: (b, 0, 0)),
        pl.BlockSpec((1, MT, 1), lambda b, mc: (b, mc, 0)),
    ],
    out_specs=[
        pl.BlockSpec((1, GPC, N, SEG), lambda b, mc: (b, mc, 0, 0)),
        pl.BlockSpec((1, N, K), lambda b, mc: (b, 0, 0)),
    ],
    out_shape=[
        jax.ShapeDtypeStruct((B, G, N, SEG), jnp.float32),
        jax.ShapeDtypeStruct((B, N, K), jnp.int32),
    ],
    scratch_shapes=[pltpu.VMEM((G, N), jnp.float32)],
)


NW = 32


def _make_sc_gather(rows_total, width, chunk):
    ni = rows_total // (NW * chunk)

    def body(tab_ref, idx_ref, out_ref, idx_c, rows_v, sem):
        c = lax.axis_index("c")
        s = lax.axis_index("s")
        base = (s * 2 + c) * (rows_total // NW)

        def step(i, carry):
            off = base + i * chunk
            pltpu.sync_copy(idx_ref.at[pl.ds(off, chunk)], idx_c)
            pltpu.async_copy(tab_ref.at[idx_c], rows_v, sem).wait()
            pltpu.sync_copy(rows_v, out_ref.at[pl.ds(off, chunk)])
            return carry

        lax.fori_loop(0, ni, step, 0)

    return pl.kernel(
        body,
        out_type=jax.ShapeDtypeStruct((rows_total, width), jnp.float32),
        mesh=plsc.VectorSubcoreMesh(core_axis_name="c", subcore_axis_name="s"),
        scratch_types=[
            pltpu.VMEM((chunk,), jnp.int32),
            pltpu.VMEM((chunk, width), jnp.float32),
            pltpu.SemaphoreType.DMA,
        ],
    )


_seg_gather = _make_sc_gather(B * N * K, SEG, 128)
_row_gather = _make_sc_gather(B * N * K, C, 128)


NB3 = 256


def _cand_body(cand_ref, seg_ref, idx_ref):
    b = pl.program_id(0)
    nt = pl.program_id(1)
    cand = cand_ref[0]
    seg = seg_ref[0]
    niota = nt * NB3 + lax.broadcasted_iota(jnp.int32, (NB3, K), 0)
    gseg = (seg - b * (G * N) - niota) // N
    ebase = gseg * SEG + b * M
    ciota = lax.broadcasted_iota(jnp.int32, (NB3, CAND), 1)
    outs = []
    vprev = jnp.full((NB3,), jnp.inf, jnp.float32)
    pprev = jnp.full((NB3,), -1, jnp.int32)
    for _ in range(K):
        elig = (cand < vprev[:, None]) | (
            (cand == vprev[:, None]) & (ciota > pprev[:, None]))
        masked = jnp.where(elig, cand, -jnp.inf)
        m = jnp.max(masked, axis=1)
        pos = jnp.min(jnp.where(masked == m[:, None], ciota, CAND), axis=1)
        slot = pos // SEG
        gsel = pos % SEG
        for s in range(K):
            gsel = gsel + jnp.where(slot == s, ebase[:, s], 0)
        outs.append(gsel)
        vprev, pprev = m, pos
    idx_ref[0] = jnp.stack(outs, axis=1)


_candtop = pl.pallas_call(
    _cand_body,
    grid=(B, N // NB3),
    in_specs=[
        pl.BlockSpec((1, NB3, CAND), lambda b, nt: (b, nt, 0)),
        pl.BlockSpec((1, NB3, K), lambda b, nt: (b, nt, 0)),
    ],
    out_specs=pl.BlockSpec((1, NB3, K), lambda b, nt: (b, nt, 0)),
    out_shape=jax.ShapeDtypeStruct((B, N, K), jnp.int32),
)


CT = 512


def _tr_body(g_ref, o_ref):
    o_ref[0] = jnp.swapaxes(g_ref[0], 0, 1)


_transpose = pl.pallas_call(
    _tr_body,
    grid=(B, (K * C) // CT),
    in_specs=[pl.BlockSpec((1, N, CT), lambda b, t: (b, 0, t))],
    out_specs=pl.BlockSpec((1, CT, N), lambda b, t: (b, t, 0)),
    out_shape=jax.ShapeDtypeStruct((B, K * C, N), jnp.float32),
)


def kernel(target_protos, ref_protos, k):
    del k
    tss = jnp.sum(target_protos * target_protos, axis=2, keepdims=True)
    rss = jnp.sum(ref_protos * ref_protos, axis=2, keepdims=True)
    sim, seg = _simtop(target_protos, ref_protos, tss, rss)
    cand = _seg_gather(sim.reshape(B * G * N, SEG), seg.reshape(-1))
    idx = _candtop(cand.reshape(B, N, CAND), seg)
    gathered = _row_gather(ref_protos.reshape(B * M, C), idx.reshape(-1))
    out = _transpose(gathered.reshape(B, N, K * C))
    return out.reshape(B, K, C, N)


# speedup vs baseline: 1.1379x; 1.1379x over previous
import functools

import jax
import jax.numpy as jnp
from jax import lax
from jax.experimental import pallas as pl
from jax.experimental.pallas import tpu as pltpu
from jax.experimental.pallas import tpu_sc as plsc

B, N, M, C, K = 4, 1024, 8192, 512, 16
MC = 8
MT = M // MC
SEG = 128
G = M // SEG
GPC = MT // SEG
CAND = K * SEG


def _sim_body(t_ref, r_ref, tss_ref, rss_ref, sim_ref, seg_ref, gv_s):
    b = pl.program_id(0)
    mc = pl.program_id(1)

    t = t_ref[0]
    r = r_ref[0]
    tn = t / (jnp.sqrt(tss_ref[0]) + 1e-8)
    rn = r / (jnp.sqrt(rss_ref[0]) + 1e-8)
    for g in range(GPC):
        sim_g = lax.dot_general(tn, rn[g * SEG:(g + 1) * SEG],
                                (((1,), (1,)), ((), ())),
                                preferred_element_type=jnp.float32)
        sim_ref[0, g] = sim_g
        gv_s[mc * GPC + g, :] = jnp.max(sim_g, axis=1)

    @pl.when(mc == MC - 1)
    def _final():
        vcur = gv_s[...]
        giota = lax.broadcasted_iota(jnp.int32, (G, N), 0)
        picks = []
        for _ in range(K):
            m = jnp.max(vcur, axis=0)
            ismax = vcur == m[None, :]
            pos = jnp.min(jnp.where(ismax, giota, G), axis=0)
            picks.append(pos)
            vcur = jnp.where(giota == pos[None, :], -jnp.inf, vcur)
        outs = []
        big = jnp.int32(G)
        for _ in range(K):
            mn = picks[0]
            for p in picks[1:]:
                mn = jnp.minimum(mn, p)
            outs.append(mn)
            picks = [jnp.where(p == mn, big, p) for p in picks]
        niota = lax.broadcasted_iota(jnp.int32, (N, K), 0)
        seg_ref[0] = jnp.stack(outs, axis=1) * N + niota + b * (G * N)


_simtop = pl.pallas_call(
    _sim_body,
    grid=(B, MC),
    in_specs=[
        pl.BlockSpec((1, N, C), lambda b, mc: (b, 0, 0)),
        pl.BlockSpec((1, MT, C), lambda b, mc: (b, mc, 0)),
        pl.BlockSpec((1, N, 1), lambda b, mc: (b, 0, 0)),
        pl.BlockSpec((1, MT, 1), lambda b, mc: (b, mc, 0)),
    ],
    out_specs=[
        pl.BlockSpec((1, GPC, N, SEG), lambda b, mc: (b, mc, 0, 0)),
        pl.BlockSpec((1, N, K), lambda b, mc: (b, 0, 0)),
    ],
    out_shape=[
        jax.ShapeDtypeStruct((B, G, N, SEG), jnp.float32),
        jax.ShapeDtypeStruct((B, N, K), jnp.int32),
    ],
    scratch_shapes=[pltpu.VMEM((G, N), jnp.float32)],
)


NW = 32


def _make_sc_gather(rows_total, width, chunk, nbuf):
    per_w = rows_total // NW
    nout = per_w // (chunk * nbuf)

    def body(tab_ref, idx_ref, out_ref, *scr):
        idx_bufs = scr[0:nbuf]
        row_bufs = scr[nbuf:2 * nbuf]
        sems = scr[2 * nbuf:3 * nbuf]
        c = lax.axis_index("c")
        s = lax.axis_index("s")
        base = (s * 2 + c) * per_w

        def outer(i, carry):
            cps = []
            for p in range(nbuf):
                off = base + (i * nbuf + p) * chunk
                pltpu.sync_copy(idx_ref.at[pl.ds(off, chunk)], idx_bufs[p])
                cps.append(pltpu.async_copy(tab_ref.at[idx_bufs[p]],
                                            row_bufs[p], sems[p]))
            for p in range(nbuf):
                off = base + (i * nbuf + p) * chunk
                cps[p].wait()
                pltpu.sync_copy(row_bufs[p], out_ref.at[pl.ds(off, chunk)])
            return carry

        lax.fori_loop(0, nout, outer, 0)

    return pl.kernel(
        body,
        out_type=jax.ShapeDtypeStruct((rows_total, width), jnp.float32),
        mesh=plsc.VectorSubcoreMesh(core_axis_name="c", subcore_axis_name="s"),
        scratch_types=(
            [pltpu.VMEM((chunk,), jnp.int32) for _ in range(nbuf)]
            + [pltpu.VMEM((chunk, width), jnp.float32) for _ in range(nbuf)]
            + [pltpu.SemaphoreType.DMA for _ in range(nbuf)]
        ),
    )


_seg_gather = _make_sc_gather(B * N * K, SEG, 128, 4)
_row_gather = _make_sc_gather(B * N * K, C, 64, 2)


NB3 = 256


def _cand_body(cand_ref, seg_ref, idx_ref):
    b = pl.program_id(0)
    nt = pl.program_id(1)
    cand = cand_ref[0]
    seg = seg_ref[0]
    niota = nt * NB3 + lax.broadcasted_iota(jnp.int32, (NB3, K), 0)
    gseg = (seg - b * (G * N) - niota) // N
    ebase = gseg * SEG + b * M
    ciota = lax.broadcasted_iota(jnp.int32, (NB3, CAND), 1)
    outs = []
    for _ in range(K):
        m = jnp.max(cand, axis=1)
        ismax = cand == m[:, None]
        pos = jnp.min(jnp.where(ismax, ciota, CAND), axis=1)
        slot = pos // SEG
        gsel = pos % SEG
        for s in range(K):
            gsel = gsel + jnp.where(slot == s, ebase[:, s], 0)
        outs.append(gsel)
        cand = jnp.where(ciota == pos[:, None], -jnp.inf, cand)
    idx_ref[0] = jnp.stack(outs, axis=1)


_candtop = pl.pallas_call(
    _cand_body,
    grid=(B, N // NB3),
    in_specs=[
        pl.BlockSpec((1, NB3, CAND), lambda b, nt: (b, nt, 0)),
        pl.BlockSpec((1, NB3, K), lambda b, nt: (b, nt, 0)),
    ],
    out_specs=pl.BlockSpec((1, NB3, K), lambda b, nt: (b, nt, 0)),
    out_shape=jax.ShapeDtypeStruct((B, N, K), jnp.int32),
)


CT = 512


def _tr_body(g_ref, o_ref):
    o_ref[0] = jnp.swapaxes(g_ref[0], 0, 1)


_transpose = pl.pallas_call(
    _tr_body,
    grid=(B, (K * C) // CT),
    in_specs=[pl.BlockSpec((1, N, CT), lambda b, t: (b, 0, t))],
    out_specs=pl.BlockSpec((1, CT, N), lambda b, t: (b, t, 0)),
    out_shape=jax.ShapeDtypeStruct((B, K * C, N), jnp.float32),
)


def kernel(target_protos, ref_protos, k):
    del k
    tss = jnp.sum(target_protos * target_protos, axis=2, keepdims=True)
    rss = jnp.sum(ref_protos * ref_protos, axis=2, keepdims=True)
    sim, seg = _simtop(target_protos, ref_protos, tss, rss)
    cand = _seg_gather(sim.reshape(B * G * N, SEG), seg.reshape(-1))
    idx = _candtop(cand.reshape(B, N, CAND), seg)
    gathered = _row_gather(ref_protos.reshape(B * M, C), idx.reshape(-1))
    out = _transpose(gathered.reshape(B, N, K * C))
    return out.reshape(B, K, C, N)
